# baseline (speedup 1.0000x reference)
import functools

import jax
import jax.numpy as jnp
from jax import lax
from jax.experimental import pallas as pl
from jax.experimental.pallas import tpu as pltpu
from jax.experimental.pallas import tpu_sc as plsc

N = 10000
E = 320000
H = 128
CLS = 40
LAYERS = 15
DT = 0.1
JACOBI = 2

NS = 16
EPT = E // NS
EC = 40
NCHUNK = EPT // EC
NSLOT = 4
NGROUP = NCHUNK // NSLOT
RPT = 624
RC = 16

BLK = 1000


def _spmv2_body(g2, src2, dst, zrows, out2,
                src_c, dst_c, rows, acc, isems, dsems, gsems, csems):
    c = lax.axis_index("c")
    s = lax.axis_index("s")
    row0 = s * RPT
    nrch = jnp.where(s == NS - 1, (N - (NS - 1) * RPT) // RC, RPT // RC)
    ebase = c * E + s * EPT
    dbase = s * EPT

    def sstart(k, j):
        pltpu.make_async_copy(
            src2.at[pl.ds(ebase + k * EC, EC)], src_c[j], isems[j]).start()

    def swait(j):
        pltpu.make_async_copy(src2.at[pl.ds(0, EC)], src_c[j], isems[j]).wait()

    def dstart(k, j):
        pltpu.make_async_copy(
            dst.at[pl.ds(dbase + k * EC, EC)], dst_c[j], dsems[j]).start()

    def dwait(j):
        pltpu.make_async_copy(dst.at[pl.ds(0, EC)], dst_c[j], dsems[j]).wait()

    for j in range(NSLOT):
        sstart(j, j)

    def zbody(j, carry):
        pltpu.sync_copy(zrows, acc.at[pl.ds(row0 + j * RC, RC)])
        return carry

    lax.fori_loop(0, nrch, zbody, 0)
    plsc.subcore_barrier()

    def body(i, carry):
        base = i * NSLOT
        for j in range(NSLOT):
            dstart(base + j, j)
            swait(j)
            pltpu.make_async_copy(g2.at[src_c[j]], rows[j], gsems[j]).start()
        for j in range(NSLOT):
            pltpu.make_async_copy(g2.at[src_c[j]], rows[j], gsems[j]).wait()

            @pl.when(i < NGROUP - 1)
            def _():
                sstart(base + NSLOT + j, j)

            dwait(j)
            pltpu.sync_copy(rows[j], acc.at[dst_c[j]], add=True)
        return carry

    lax.fori_loop(0, NGROUP, body, 0)
    plsc.subcore_barrier()
    ob = c * N + row0

    def wbody(j, carry):
        pltpu.sync_copy(acc.at[pl.ds(row0 + j * RC, RC)],
                        out2.at[pl.ds(ob + j * RC, RC)])
        return carry

    lax.fori_loop(0, nrch, wbody, 0)


_spmv2 = pl.kernel(
    _spmv2_body,
    out_type=jax.ShapeDtypeStruct((2 * N, H), jnp.float32),
    mesh=plsc.VectorSubcoreMesh(core_axis_name="c", subcore_axis_name="s"),
    scratch_types=[
        [pltpu.VMEM((EC,), jnp.int32) for _ in range(NSLOT)],
        [pltpu.VMEM((EC,), jnp.int32) for _ in range(NSLOT)],
        [pltpu.VMEM((EC, H), jnp.float32) for _ in range(NSLOT)],
        pltpu.VMEM_SHARED((N, H), jnp.float32),
        [pltpu.SemaphoreType.DMA for _ in range(NSLOT)],
        [pltpu.SemaphoreType.DMA for _ in range(NSLOT)],
        [pltpu.SemaphoreType.DMA for _ in range(NSLOT)],
        [pltpu.SemaphoreType.DMA for _ in range(NSLOT)],
    ],
)


def _lift_body(x_ref, w_ref, b_ref, o_ref):
    o_ref[...] = jnp.tanh(
        jnp.dot(x_ref[...], w_ref[...], preferred_element_type=jnp.float32)
        + b_ref[...])


def _readout_body(x_ref, w_ref, b_ref, o_ref):
    o_ref[...] = (
        jnp.dot(x_ref[...], w_ref[...], preferred_element_type=jnp.float32)
        + b_ref[...])


def _stage_a_body(x_ref, y_ref, d_ref, r2_ref, g2_ref):
    x = x_ref[...]
    y = y_ref[...]
    d = d_ref[...]
    rx = x + DT * (x * (1.0 - y))
    ry = y + DT * (y * (x - 1.0))
    r2_ref[0] = rx
    r2_ref[1] = ry
    g2_ref[0] = d * rx
    g2_ref[1] = d * ry


def _stage_b_body(g2_ref, q2_ref, d_ref, w2_ref):
    d = d_ref[...]
    scale = 1.0 / (1.0 + DT)
    dd = d * d
    w2_ref[0] = (g2_ref[0] + DT * dd * q2_ref[0]) * scale
    w2_ref[1] = (g2_ref[1] + DT * dd * q2_ref[1]) * scale


def _stage_c_body(tau_ref, x_ref, y_ref, r2_ref, q2_ref, d_ref,
                  xo_ref, yo_ref):
    d = d_ref[...]
    tau = tau_ref[0]
    scale = 1.0 / (1.0 + DT)
    zx = (r2_ref[0] + DT * d * q2_ref[0]) * scale
    zy = (r2_ref[1] + DT * d * q2_ref[1]) * scale
    xo_ref[...] = (1.0 - tau) * x_ref[...] + tau * zx
    yo_ref[...] = (1.0 - tau) * y_ref[...] + tau * zy


_GRID = N // BLK
_row_spec = pl.BlockSpec((BLK, H), lambda i: (i, 0))
_pair_spec = pl.BlockSpec((2, BLK, H), lambda i: (0, i, 0))
_full_spec = pl.BlockSpec((H, H), lambda i: (0, 0))
_brow_spec = pl.BlockSpec((1, H), lambda i: (0, 0))

_lift = pl.pallas_call(
    _lift_body,
    grid=(_GRID,),
    in_specs=[_row_spec, _full_spec, _brow_spec],
    out_specs=_row_spec,
    out_shape=jax.ShapeDtypeStruct((N, H), jnp.float32),
)

_readout = pl.pallas_call(
    _readout_body,
    grid=(_GRID,),
    in_specs=[_row_spec, _full_spec, _brow_spec],
    out_specs=_row_spec,
    out_shape=jax.ShapeDtypeStruct((N, H), jnp.float32),
)

_stage_a = pl.pallas_call(
    _stage_a_body,
    grid=(_GRID,),
    in_specs=[_row_spec, _row_spec, _row_spec],
    out_specs=[_pair_spec, _pair_spec],
    out_shape=[jax.ShapeDtypeStruct((2, N, H), jnp.float32),
               jax.ShapeDtypeStruct((2, N, H), jnp.float32)],
)

_stage_b = pl.pallas_call(
    _stage_b_body,
    grid=(_GRID,),
    in_specs=[_pair_spec, _pair_spec, _row_spec],
    out_specs=_pair_spec,
    out_shape=jax.ShapeDtypeStruct((2, N, H), jnp.float32),
)

_stage_c = pl.pallas_call(
    _stage_c_body,
    grid=(_GRID,),
    in_specs=[pl.BlockSpec(memory_space=pltpu.SMEM),
              _row_spec, _row_spec, _pair_spec, _pair_spec, _row_spec],
    out_specs=[_row_spec, _row_spec],
    out_shape=[jax.ShapeDtypeStruct((N, H), jnp.float32),
               jax.ShapeDtypeStruct((N, H), jnp.float32)],
)


def kernel(x, edge_index, lift_x_w, lift_x_b, taus, logit_scale,
           readout_w, readout_b):
    src = edge_index[0]
    dst = edge_index[1]
    src2 = jnp.concatenate([src, src + N])


    ones2 = jnp.ones((2 * N, H), dtype=jnp.float32)
    zrows = jnp.zeros((RC, H), dtype=jnp.float32)
    deg = _spmv2(ones2, src2, dst, zrows)[:N, :1]
    dinv = 1.0 / jnp.sqrt(jnp.maximum(deg, 1.0))
    dinvb = jnp.broadcast_to(dinv, (N, H))

    X = _lift(x, lift_x_w, lift_x_b.reshape(1, H))
    Y = jnp.ones_like(X)

    sig_taus = jax.nn.sigmoid(taus)

    for li in range(LAYERS):
        r2, g2 = _stage_a(X, Y, dinvb)
        q2 = _spmv2(g2.reshape(2 * N, H), src2, dst, zrows)
        w2 = _stage_b(g2, q2.reshape(2, N, H), dinvb)
        q2b = _spmv2(w2.reshape(2 * N, H), src2, dst, zrows)
        X, Y = _stage_c(sig_taus[li].reshape(1), X, Y, r2,
                        q2b.reshape(2, N, H), dinvb)

    wp = jnp.zeros((H, H), jnp.float32).at[:, :CLS].set(
        logit_scale * readout_w)
    bp = jnp.zeros((1, H), jnp.float32).at[0, :CLS].set(readout_b)
    out = _readout(X, wp, bp)
    return out[:, :CLS]


# speedup vs baseline: 1.2326x; 1.2326x over previous
import functools

import jax
import jax.numpy as jnp
from jax import lax
from jax.experimental import pallas as pl
from jax.experimental.pallas import tpu as pltpu
from jax.experimental.pallas import tpu_sc as plsc

N = 10000
E = 320000
H = 128
CLS = 40
LAYERS = 15
DT = 0.1
JACOBI = 2

NS = 16
EPT = E // NS
EC = 80
NCHUNK = EPT // EC
NSLOT = 2
NGROUP = NCHUNK // NSLOT
RPT = 624
RC = 16

BLK = 1000


def _spmv2_body(g2, src2, dst, zrows, out2,
                src_c, dst_c, rows, acc, isems, dsems, gsems, csems):
    c = lax.axis_index("c")
    s = lax.axis_index("s")
    row0 = s * RPT
    nrch = jnp.where(s == NS - 1, (N - (NS - 1) * RPT) // RC, RPT // RC)
    ebase = c * E + s * EPT
    dbase = s * EPT

    def sstart(k, j):
        pltpu.make_async_copy(
            src2.at[pl.ds(ebase + k * EC, EC)], src_c[j], isems[j]).start()

    def swait(j):
        pltpu.make_async_copy(src2.at[pl.ds(0, EC)], src_c[j], isems[j]).wait()

    def dstart(k, j):
        pltpu.make_async_copy(
            dst.at[pl.ds(dbase + k * EC, EC)], dst_c[j], dsems[j]).start()

    def dwait(j):
        pltpu.make_async_copy(dst.at[pl.ds(0, EC)], dst_c[j], dsems[j]).wait()

    for j in range(NSLOT):
        sstart(j, j)

    def zbody(j, carry):
        pltpu.sync_copy(zrows, acc.at[pl.ds(row0 + j * RC, RC)])
        return carry

    lax.fori_loop(0, nrch, zbody, 0)
    plsc.subcore_barrier()

    def gstart(j):
        pltpu.make_async_copy(g2.at[src_c[j]], rows[j], gsems[j]).start()

    def gwait(j):
        pltpu.make_async_copy(g2.at[src_c[j]], rows[j], gsems[j]).wait()

    def cstart(j):
        pltpu.async_copy(rows[j], acc.at[dst_c[j]], csems[j], add=True)

    def cwait(j):
        pltpu.make_async_copy(rows[j], acc.at[dst_c[j]], csems[j]).wait()

    def body(i, carry):
        k0 = NSLOT * i
        dstart(k0, 0)
        swait(0)
        gstart(0)

        @pl.when(i > 0)
        def _():
            cwait(1)

        dstart(k0 + 1, 1)
        swait(1)
        gstart(1)

        gwait(0)

        @pl.when(i < NGROUP - 1)
        def _():
            sstart(k0 + NSLOT, 0)

        dwait(0)
        cstart(0)

        gwait(1)

        @pl.when(i < NGROUP - 1)
        def _():
            sstart(k0 + NSLOT + 1, 1)

        dwait(1)
        cwait(0)
        cstart(1)
        return carry

    lax.fori_loop(0, NGROUP, body, 0)
    cwait(1)
    plsc.subcore_barrier()
    ob = c * N + row0

    def wbody(j, carry):
        pltpu.sync_copy(acc.at[pl.ds(row0 + j * RC, RC)],
                        out2.at[pl.ds(ob + j * RC, RC)])
        return carry

    lax.fori_loop(0, nrch, wbody, 0)


_spmv2 = pl.kernel(
    _spmv2_body,
    out_type=jax.ShapeDtypeStruct((2 * N, H), jnp.float32),
    mesh=plsc.VectorSubcoreMesh(core_axis_name="c", subcore_axis_name="s"),
    scratch_types=[
        [pltpu.VMEM((EC,), jnp.int32) for _ in range(NSLOT)],
        [pltpu.VMEM((EC,), jnp.int32) for _ in range(NSLOT)],
        [pltpu.VMEM((EC, H), jnp.float32) for _ in range(NSLOT)],
        pltpu.VMEM_SHARED((N, H), jnp.float32),
        [pltpu.SemaphoreType.DMA for _ in range(NSLOT)],
        [pltpu.SemaphoreType.DMA for _ in range(NSLOT)],
        [pltpu.SemaphoreType.DMA for _ in range(NSLOT)],
        [pltpu.SemaphoreType.DMA for _ in range(NSLOT)],
    ],
)


def _lift_body(x_ref, w_ref, b_ref, o_ref):
    o_ref[...] = jnp.tanh(
        jnp.dot(x_ref[...], w_ref[...], preferred_element_type=jnp.float32)
        + b_ref[...])


def _readout_body(x_ref, w_ref, b_ref, o_ref):
    o_ref[...] = (
        jnp.dot(x_ref[...], w_ref[...], preferred_element_type=jnp.float32)
        + b_ref[...])


def _stage_a_body(x_ref, y_ref, d_ref, r2_ref, g2_ref):
    x = x_ref[...]
    y = y_ref[...]
    d = d_ref[...]
    rx = x + DT * (x * (1.0 - y))
    ry = y + DT * (y * (x - 1.0))
    r2_ref[0] = rx
    r2_ref[1] = ry
    g2_ref[0] = d * rx
    g2_ref[1] = d * ry


def _stage_b_body(g2_ref, q2_ref, d_ref, w2_ref):
    d = d_ref[...]
    scale = 1.0 / (1.0 + DT)
    dd = d * d
    w2_ref[0] = (g2_ref[0] + DT * dd * q2_ref[0]) * scale
    w2_ref[1] = (g2_ref[1] + DT * dd * q2_ref[1]) * scale


def _stage_c_body(tau_ref, x_ref, y_ref, r2_ref, q2_ref, d_ref,
                  xo_ref, yo_ref):
    d = d_ref[...]
    tau = tau_ref[0]
    scale = 1.0 / (1.0 + DT)
    zx = (r2_ref[0] + DT * d * q2_ref[0]) * scale
    zy = (r2_ref[1] + DT * d * q2_ref[1]) * scale
    xo_ref[...] = (1.0 - tau) * x_ref[...] + tau * zx
    yo_ref[...] = (1.0 - tau) * y_ref[...] + tau * zy


_GRID = N // BLK
_row_spec = pl.BlockSpec((BLK, H), lambda i: (i, 0))
_pair_spec = pl.BlockSpec((2, BLK, H), lambda i: (0, i, 0))
_full_spec = pl.BlockSpec((H, H), lambda i: (0, 0))
_brow_spec = pl.BlockSpec((1, H), lambda i: (0, 0))

_lift = pl.pallas_call(
    _lift_body,
    grid=(_GRID,),
    in_specs=[_row_spec, _full_spec, _brow_spec],
    out_specs=_row_spec,
    out_shape=jax.ShapeDtypeStruct((N, H), jnp.float32),
)

_readout = pl.pallas_call(
    _readout_body,
    grid=(_GRID,),
    in_specs=[_row_spec, _full_spec, _brow_spec],
    out_specs=_row_spec,
    out_shape=jax.ShapeDtypeStruct((N, H), jnp.float32),
)

_stage_a = pl.pallas_call(
    _stage_a_body,
    grid=(_GRID,),
    in_specs=[_row_spec, _row_spec, _row_spec],
    out_specs=[_pair_spec, _pair_spec],
    out_shape=[jax.ShapeDtypeStruct((2, N, H), jnp.float32),
               jax.ShapeDtypeStruct((2, N, H), jnp.float32)],
)

_stage_b = pl.pallas_call(
    _stage_b_body,
    grid=(_GRID,),
    in_specs=[_pair_spec, _pair_spec, _row_spec],
    out_specs=_pair_spec,
    out_shape=jax.ShapeDtypeStruct((2, N, H), jnp.float32),
)

_stage_c = pl.pallas_call(
    _stage_c_body,
    grid=(_GRID,),
    in_specs=[pl.BlockSpec(memory_space=pltpu.SMEM),
              _row_spec, _row_spec, _pair_spec, _pair_spec, _row_spec],
    out_specs=[_row_spec, _row_spec],
    out_shape=[jax.ShapeDtypeStruct((N, H), jnp.float32),
               jax.ShapeDtypeStruct((N, H), jnp.float32)],
)


def kernel(x, edge_index, lift_x_w, lift_x_b, taus, logit_scale,
           readout_w, readout_b):
    src = edge_index[0]
    dst = edge_index[1]
    src2 = jnp.concatenate([src, src + N])


    ones2 = jnp.ones((2 * N, H), dtype=jnp.float32)
    zrows = jnp.zeros((RC, H), dtype=jnp.float32)
    deg = _spmv2(ones2, src2, dst, zrows)[:N, :1]
    dinv = 1.0 / jnp.sqrt(jnp.maximum(deg, 1.0))
    dinvb = jnp.broadcast_to(dinv, (N, H))

    X = _lift(x, lift_x_w, lift_x_b.reshape(1, H))
    Y = jnp.ones_like(X)

    sig_taus = jax.nn.sigmoid(taus)

    for li in range(LAYERS):
        r2, g2 = _stage_a(X, Y, dinvb)
        q2 = _spmv2(g2.reshape(2 * N, H), src2, dst, zrows)
        w2 = _stage_b(g2, q2.reshape(2, N, H), dinvb)
        q2b = _spmv2(w2.reshape(2 * N, H), src2, dst, zrows)
        X, Y = _stage_c(sig_taus[li].reshape(1), X, Y, r2,
                        q2b.reshape(2, N, H), dinvb)

    wp = jnp.zeros((H, H), jnp.float32).at[:, :CLS].set(
        logit_scale * readout_w)
    bp = jnp.zeros((1, H), jnp.float32).at[0, :CLS].set(readout_b)
    out = _readout(X, wp, bp)
    return out[:, :CLS]


# speedup vs baseline: 1.5163x; 1.2301x over previous
import functools

import jax
import jax.numpy as jnp
from jax import lax
from jax.experimental import pallas as pl
from jax.experimental.pallas import tpu as pltpu
from jax.experimental.pallas import tpu_sc as plsc

N = 10000
E = 320000
H = 128
CLS = 40
LAYERS = 15
DT = 0.1
JACOBI = 2

NS = 16
EPT = E // NS
EC = 80
NCHUNK = EPT // EC
NSLOT = 2
NGROUP = NCHUNK // NSLOT
RPT = 624
RC = 16

BLK = 1000


def _spmv2_body(g2, src2, dst, zrows, out2,
                src_c, dst_c, rows, acc, isems, dsems, gsems, csems):
    c = lax.axis_index("c")
    s = lax.axis_index("s")
    row0 = s * RPT
    ebase = c * E + s * EPT
    dbase = s * EPT
    tail = NS * RPT

    def sstart(k, j):
        pltpu.make_async_copy(
            src2.at[pl.ds(ebase + k * EC, EC)], src_c[j], isems[j]).start()

    def swait(j):
        pltpu.make_async_copy(src2.at[pl.ds(0, EC)], src_c[j], isems[j]).wait()

    def dstart(k, j):
        pltpu.make_async_copy(
            dst.at[pl.ds(dbase + k * EC, EC)], dst_c[j], dsems[j]).start()

    def dwait(j):
        pltpu.make_async_copy(dst.at[pl.ds(0, EC)], dst_c[j], dsems[j]).wait()

    for j in range(NSLOT):
        sstart(j, j)

    pltpu.sync_copy(zrows, acc.at[pl.ds(row0, RPT)])

    @pl.when(s == NS - 1)
    def _():
        pltpu.sync_copy(zrows.at[pl.ds(0, N - NS * RPT)],
                        acc.at[pl.ds(tail, N - NS * RPT)])

    plsc.subcore_barrier()

    def gstart(j):
        pltpu.make_async_copy(g2.at[src_c[j]], rows[j], gsems[j]).start()

    def gwait(j):
        pltpu.make_async_copy(g2.at[src_c[j]], rows[j], gsems[j]).wait()

    def cstart(j):
        pltpu.async_copy(rows[j], acc.at[dst_c[j]], csems[j], add=True)

    def cwait(j):
        pltpu.make_async_copy(rows[j], acc.at[dst_c[j]], csems[j]).wait()

    def body(i, carry):
        k0 = NSLOT * i
        dstart(k0, 0)
        swait(0)
        gstart(0)

        @pl.when(i > 0)
        def _():
            cwait(1)

        dstart(k0 + 1, 1)
        swait(1)
        gstart(1)

        gwait(0)

        @pl.when(i < NGROUP - 1)
        def _():
            sstart(k0 + NSLOT, 0)

        dwait(0)
        cstart(0)

        gwait(1)

        @pl.when(i < NGROUP - 1)
        def _():
            sstart(k0 + NSLOT + 1, 1)

        dwait(1)
        cwait(0)
        cstart(1)
        return carry

    lax.fori_loop(0, NGROUP, body, 0)
    cwait(1)
    plsc.subcore_barrier()
    ob = c * N + row0
    pltpu.sync_copy(acc.at[pl.ds(row0, RPT)], out2.at[pl.ds(ob, RPT)])

    @pl.when(s == NS - 1)
    def _():
        pltpu.sync_copy(acc.at[pl.ds(tail, N - NS * RPT)],
                        out2.at[pl.ds(c * N + tail, N - NS * RPT)])


_spmv2 = pl.kernel(
    _spmv2_body,
    out_type=jax.ShapeDtypeStruct((2 * N, H), jnp.float32),
    mesh=plsc.VectorSubcoreMesh(core_axis_name="c", subcore_axis_name="s"),
    scratch_types=[
        [pltpu.VMEM((EC,), jnp.int32) for _ in range(NSLOT)],
        [pltpu.VMEM((EC,), jnp.int32) for _ in range(NSLOT)],
        [pltpu.VMEM((EC, H), jnp.float32) for _ in range(NSLOT)],
        pltpu.VMEM_SHARED((N, H), jnp.float32),
        [pltpu.SemaphoreType.DMA for _ in range(NSLOT)],
        [pltpu.SemaphoreType.DMA for _ in range(NSLOT)],
        [pltpu.SemaphoreType.DMA for _ in range(NSLOT)],
        [pltpu.SemaphoreType.DMA for _ in range(NSLOT)],
    ],
)


def _lift_body(x_ref, w_ref, b_ref, o_ref):
    o_ref[...] = jnp.tanh(
        jnp.dot(x_ref[...], w_ref[...], preferred_element_type=jnp.float32)
        + b_ref[...])


def _readout_body(x_ref, w_ref, b_ref, o_ref):
    o_ref[...] = (
        jnp.dot(x_ref[...], w_ref[...], preferred_element_type=jnp.float32)
        + b_ref[...])


def _stage_a_body(x_ref, y_ref, d_ref, r2_ref, g2_ref):
    x = x_ref[...]
    y = y_ref[...]
    d = d_ref[...]
    rx = x + DT * (x * (1.0 - y))
    ry = y + DT * (y * (x - 1.0))
    r2_ref[0] = rx
    r2_ref[1] = ry
    g2_ref[0] = d * rx
    g2_ref[1] = d * ry


def _stage_b_body(g2_ref, q2_ref, d_ref, w2_ref):
    d = d_ref[...]
    scale = 1.0 / (1.0 + DT)
    dd = d * d
    w2_ref[0] = (g2_ref[0] + DT * dd * q2_ref[0]) * scale
    w2_ref[1] = (g2_ref[1] + DT * dd * q2_ref[1]) * scale


def _stage_c_body(tau_ref, x_ref, y_ref, r2_ref, q2_ref, d_ref,
                  xo_ref, yo_ref):
    d = d_ref[...]
    tau = tau_ref[0]
    scale = 1.0 / (1.0 + DT)
    zx = (r2_ref[0] + DT * d * q2_ref[0]) * scale
    zy = (r2_ref[1] + DT * d * q2_ref[1]) * scale
    xo_ref[...] = (1.0 - tau) * x_ref[...] + tau * zx
    yo_ref[...] = (1.0 - tau) * y_ref[...] + tau * zy


_GRID = N // BLK
_row_spec = pl.BlockSpec((BLK, H), lambda i: (i, 0))
_pair_spec = pl.BlockSpec((2, BLK, H), lambda i: (0, i, 0))
_full_spec = pl.BlockSpec((H, H), lambda i: (0, 0))
_brow_spec = pl.BlockSpec((1, H), lambda i: (0, 0))

_lift = pl.pallas_call(
    _lift_body,
    grid=(_GRID,),
    in_specs=[_row_spec, _full_spec, _brow_spec],
    out_specs=_row_spec,
    out_shape=jax.ShapeDtypeStruct((N, H), jnp.float32),
)

_readout = pl.pallas_call(
    _readout_body,
    grid=(_GRID,),
    in_specs=[_row_spec, _full_spec, _brow_spec],
    out_specs=_row_spec,
    out_shape=jax.ShapeDtypeStruct((N, H), jnp.float32),
)

_stage_a = pl.pallas_call(
    _stage_a_body,
    grid=(_GRID,),
    in_specs=[_row_spec, _row_spec, _row_spec],
    out_specs=[_pair_spec, _pair_spec],
    out_shape=[jax.ShapeDtypeStruct((2, N, H), jnp.float32),
               jax.ShapeDtypeStruct((2, N, H), jnp.float32)],
)

_stage_b = pl.pallas_call(
    _stage_b_body,
    grid=(_GRID,),
    in_specs=[_pair_spec, _pair_spec, _row_spec],
    out_specs=_pair_spec,
    out_shape=jax.ShapeDtypeStruct((2, N, H), jnp.float32),
)

_stage_c = pl.pallas_call(
    _stage_c_body,
    grid=(_GRID,),
    in_specs=[pl.BlockSpec(memory_space=pltpu.SMEM),
              _row_spec, _row_spec, _pair_spec, _pair_spec, _row_spec],
    out_specs=[_row_spec, _row_spec],
    out_shape=[jax.ShapeDtypeStruct((N, H), jnp.float32),
               jax.ShapeDtypeStruct((N, H), jnp.float32)],
)


def kernel(x, edge_index, lift_x_w, lift_x_b, taus, logit_scale,
           readout_w, readout_b):
    src = edge_index[0]
    dst = edge_index[1]
    src2 = jnp.concatenate([src, src + N])


    ones2 = jnp.ones((2 * N, H), dtype=jnp.float32)
    zrows = jnp.zeros((RPT, H), dtype=jnp.float32)
    deg = _spmv2(ones2, src2, dst, zrows)[:N, :1]
    dinv = 1.0 / jnp.sqrt(jnp.maximum(deg, 1.0))
    dinvb = jnp.broadcast_to(dinv, (N, H))

    X = _lift(x, lift_x_w, lift_x_b.reshape(1, H))
    Y = jnp.ones_like(X)

    sig_taus = jax.nn.sigmoid(taus)

    for li in range(LAYERS):
        r2, g2 = _stage_a(X, Y, dinvb)
        q2 = _spmv2(g2.reshape(2 * N, H), src2, dst, zrows)
        w2 = _stage_b(g2, q2.reshape(2, N, H), dinvb)
        q2b = _spmv2(w2.reshape(2 * N, H), src2, dst, zrows)
        X, Y = _stage_c(sig_taus[li].reshape(1), X, Y, r2,
                        q2b.reshape(2, N, H), dinvb)

    wp = jnp.zeros((H, H), jnp.float32).at[:, :CLS].set(
        logit_scale * readout_w)
    bp = jnp.zeros((1, H), jnp.float32).at[0, :CLS].set(readout_b)
    out = _readout(X, wp, bp)
    return out[:, :CLS]


# speedup vs baseline: 1.5341x; 1.0118x over previous
import functools

import jax
import jax.numpy as jnp
from jax import lax
from jax.experimental import pallas as pl
from jax.experimental.pallas import tpu as pltpu
from jax.experimental.pallas import tpu_sc as plsc

N = 10000
E = 320000
H = 128
CLS = 40
LAYERS = 15
DT = 0.1
JACOBI = 2

NS = 16
EPT = E // NS
EC = 80
NCHUNK = EPT // EC
NSLOT = 2
NGROUP = NCHUNK // NSLOT
RPT = 624
RC = 16

BLK = 1000


def _spmv2_body(g2, src2, dst, zrows, out2,
                src_c, dst_c, rows, acc, isems, dsems, gsems, csems):
    c = lax.axis_index("c")
    s = lax.axis_index("s")
    row0 = s * RPT
    ebase = c * E + s * EPT
    dbase = s * EPT
    tail = NS * RPT

    def sstart(k, j):
        pltpu.make_async_copy(
            src2.at[pl.ds(ebase + k * EC, EC)], src_c[j], isems[j]).start()

    def swait(j):
        pltpu.make_async_copy(src2.at[pl.ds(0, EC)], src_c[j], isems[j]).wait()

    def dstart(k, j):
        pltpu.make_async_copy(
            dst.at[pl.ds(dbase + k * EC, EC)], dst_c[j], dsems[j]).start()

    def dwait(j):
        pltpu.make_async_copy(dst.at[pl.ds(0, EC)], dst_c[j], dsems[j]).wait()

    for j in range(NSLOT):
        sstart(j, j)

    pltpu.sync_copy(zrows, acc.at[pl.ds(row0, RPT)])

    @pl.when(s == NS - 1)
    def _():
        pltpu.sync_copy(zrows.at[pl.ds(0, N - NS * RPT)],
                        acc.at[pl.ds(tail, N - NS * RPT)])

    plsc.subcore_barrier()

    def gstart(j):
        pltpu.make_async_copy(g2.at[src_c[j]], rows[j], gsems[j]).start()

    def gwait(j):
        pltpu.make_async_copy(g2.at[src_c[j]], rows[j], gsems[j]).wait()

    def cstart(j):
        pltpu.async_copy(rows[j], acc.at[dst_c[j]], csems[j], add=True)

    def cwait(j):
        pltpu.make_async_copy(rows[j], acc.at[dst_c[j]], csems[j]).wait()

    def body(i, carry):
        k0 = NSLOT * i
        dstart(k0, 0)
        swait(0)
        gstart(0)

        @pl.when(i > 0)
        def _():
            cwait(1)

        dstart(k0 + 1, 1)
        swait(1)
        gstart(1)

        gwait(0)

        @pl.when(i < NGROUP - 1)
        def _():
            sstart(k0 + NSLOT, 0)

        dwait(0)
        cstart(0)

        gwait(1)

        @pl.when(i < NGROUP - 1)
        def _():
            sstart(k0 + NSLOT + 1, 1)

        dwait(1)
        cwait(0)
        cstart(1)
        return carry

    lax.fori_loop(0, NGROUP, body, 0)
    cwait(1)
    plsc.subcore_barrier()
    ob = c * N + row0
    pltpu.sync_copy(acc.at[pl.ds(row0, RPT)], out2.at[pl.ds(ob, RPT)])

    @pl.when(s == NS - 1)
    def _():
        pltpu.sync_copy(acc.at[pl.ds(tail, N - NS * RPT)],
                        out2.at[pl.ds(c * N + tail, N - NS * RPT)])


_spmv2 = pl.kernel(
    _spmv2_body,
    out_type=jax.ShapeDtypeStruct((2 * N, H), jnp.float32),
    mesh=plsc.VectorSubcoreMesh(core_axis_name="c", subcore_axis_name="s"),
    scratch_types=[
        [pltpu.VMEM((EC,), jnp.int32) for _ in range(NSLOT)],
        [pltpu.VMEM((EC,), jnp.int32) for _ in range(NSLOT)],
        [pltpu.VMEM((EC, H), jnp.float32) for _ in range(NSLOT)],
        pltpu.VMEM_SHARED((N, H), jnp.float32),
        [pltpu.SemaphoreType.DMA for _ in range(NSLOT)],
        [pltpu.SemaphoreType.DMA for _ in range(NSLOT)],
        [pltpu.SemaphoreType.DMA for _ in range(NSLOT)],
        [pltpu.SemaphoreType.DMA for _ in range(NSLOT)],
    ],
)


def _lift_body(x_ref, w_ref, b_ref, o_ref):
    o_ref[...] = jnp.tanh(
        jnp.dot(x_ref[...], w_ref[...], preferred_element_type=jnp.float32)
        + b_ref[...])


def _readout_body(x_ref, w_ref, b_ref, o_ref):
    o_ref[...] = (
        jnp.dot(x_ref[...], w_ref[...], preferred_element_type=jnp.float32)
        + b_ref[...])


def _stage_a_body(x_ref, y_ref, d_ref, r2_ref, g2_ref):
    x = x_ref[...]
    y = y_ref[...]
    d = d_ref[...]
    rx = x + DT * (x * (1.0 - y))
    ry = y + DT * (y * (x - 1.0))
    r2_ref[0] = rx
    r2_ref[1] = ry
    g2_ref[0] = d * rx
    g2_ref[1] = d * ry


def _stage_b_body(g2_ref, q2_ref, d_ref, w2_ref):
    d = d_ref[...]
    scale = 1.0 / (1.0 + DT)
    dd = d * d
    w2_ref[0] = (g2_ref[0] + DT * dd * q2_ref[0]) * scale
    w2_ref[1] = (g2_ref[1] + DT * dd * q2_ref[1]) * scale


def _stage_c_body(tau_ref, x_ref, y_ref, r2_ref, q2_ref, d_ref,
                  xo_ref, yo_ref):
    d = d_ref[...]
    tau = tau_ref[0]
    scale = 1.0 / (1.0 + DT)
    zx = (r2_ref[0] + DT * d * q2_ref[0]) * scale
    zy = (r2_ref[1] + DT * d * q2_ref[1]) * scale
    xo_ref[...] = (1.0 - tau) * x_ref[...] + tau * zx
    yo_ref[...] = (1.0 - tau) * y_ref[...] + tau * zy


def _stage_ca_body(tau_ref, x_ref, y_ref, r2_ref, q2_ref, d_ref,
                   xo_ref, yo_ref, r2o_ref, g2o_ref):
    d = d_ref[...]
    tau = tau_ref[0]
    scale = 1.0 / (1.0 + DT)
    zx = (r2_ref[0] + DT * d * q2_ref[0]) * scale
    zy = (r2_ref[1] + DT * d * q2_ref[1]) * scale
    x = (1.0 - tau) * x_ref[...] + tau * zx
    y = (1.0 - tau) * y_ref[...] + tau * zy
    xo_ref[...] = x
    yo_ref[...] = y
    rx = x + DT * (x * (1.0 - y))
    ry = y + DT * (y * (x - 1.0))
    r2o_ref[0] = rx
    r2o_ref[1] = ry
    g2o_ref[0] = d * rx
    g2o_ref[1] = d * ry


_GRID = N // BLK
_row_spec = pl.BlockSpec((BLK, H), lambda i: (i, 0))
_pair_spec = pl.BlockSpec((2, BLK, H), lambda i: (0, i, 0))
_full_spec = pl.BlockSpec((H, H), lambda i: (0, 0))
_brow_spec = pl.BlockSpec((1, H), lambda i: (0, 0))

_lift = pl.pallas_call(
    _lift_body,
    grid=(_GRID,),
    in_specs=[_row_spec, _full_spec, _brow_spec],
    out_specs=_row_spec,
    out_shape=jax.ShapeDtypeStruct((N, H), jnp.float32),
)

_readout = pl.pallas_call(
    _readout_body,
    grid=(_GRID,),
    in_specs=[_row_spec, _full_spec, _brow_spec],
    out_specs=_row_spec,
    out_shape=jax.ShapeDtypeStruct((N, H), jnp.float32),
)

_stage_a = pl.pallas_call(
    _stage_a_body,
    grid=(_GRID,),
    in_specs=[_row_spec, _row_spec, _row_spec],
    out_specs=[_pair_spec, _pair_spec],
    out_shape=[jax.ShapeDtypeStruct((2, N, H), jnp.float32),
               jax.ShapeDtypeStruct((2, N, H), jnp.float32)],
)

_stage_b = pl.pallas_call(
    _stage_b_body,
    grid=(_GRID,),
    in_specs=[_pair_spec, _pair_spec, _row_spec],
    out_specs=_pair_spec,
    out_shape=jax.ShapeDtypeStruct((2, N, H), jnp.float32),
)

_stage_c = pl.pallas_call(
    _stage_c_body,
    grid=(_GRID,),
    in_specs=[pl.BlockSpec(memory_space=pltpu.SMEM),
              _row_spec, _row_spec, _pair_spec, _pair_spec, _row_spec],
    out_specs=[_row_spec, _row_spec],
    out_shape=[jax.ShapeDtypeStruct((N, H), jnp.float32),
               jax.ShapeDtypeStruct((N, H), jnp.float32)],
)

_stage_ca = pl.pallas_call(
    _stage_ca_body,
    grid=(_GRID,),
    in_specs=[pl.BlockSpec(memory_space=pltpu.SMEM),
              _row_spec, _row_spec, _pair_spec, _pair_spec, _row_spec],
    out_specs=[_row_spec, _row_spec, _pair_spec, _pair_spec],
    out_shape=[jax.ShapeDtypeStruct((N, H), jnp.float32),
               jax.ShapeDtypeStruct((N, H), jnp.float32),
               jax.ShapeDtypeStruct((2, N, H), jnp.float32),
               jax.ShapeDtypeStruct((2, N, H), jnp.float32)],
)


def kernel(x, edge_index, lift_x_w, lift_x_b, taus, logit_scale,
           readout_w, readout_b):
    src = edge_index[0]
    dst = edge_index[1]
    src2 = jnp.concatenate([src, src + N])


    ones2 = jnp.ones((2 * N, H), dtype=jnp.float32)
    zrows = jnp.zeros((RPT, H), dtype=jnp.float32)
    deg = _spmv2(ones2, src2, dst, zrows)[:N, :1]
    dinv = 1.0 / jnp.sqrt(jnp.maximum(deg, 1.0))
    dinvb = jnp.broadcast_to(dinv, (N, H))

    X = _lift(x, lift_x_w, lift_x_b.reshape(1, H))
    Y = jnp.ones_like(X)

    sig_taus = jax.nn.sigmoid(taus)

    r2, g2 = _stage_a(X, Y, dinvb)
    for li in range(LAYERS):
        q2 = _spmv2(g2.reshape(2 * N, H), src2, dst, zrows)
        w2 = _stage_b(g2, q2.reshape(2, N, H), dinvb)
        q2b = _spmv2(w2.reshape(2 * N, H), src2, dst, zrows)
        if li < LAYERS - 1:
            X, Y, r2, g2 = _stage_ca(sig_taus[li].reshape(1), X, Y, r2,
                                     q2b.reshape(2, N, H), dinvb)
        else:
            X, Y = _stage_c(sig_taus[li].reshape(1), X, Y, r2,
                            q2b.reshape(2, N, H), dinvb)

    wp = jnp.zeros((H, H), jnp.float32).at[:, :CLS].set(
        logit_scale * readout_w)
    bp = jnp.zeros((1, H), jnp.float32).at[0, :CLS].set(readout_b)
    out = _readout(X, wp, bp)
    return out[:, :CLS]


# speedup vs baseline: 1.6966x; 1.1059x over previous
import functools

import jax
import jax.numpy as jnp
from jax import lax
from jax.experimental import pallas as pl
from jax.experimental.pallas import tpu as pltpu
from jax.experimental.pallas import tpu_sc as plsc

N = 10000
E = 320000
H = 128
CLS = 40
LAYERS = 15
DT = 0.1
JACOBI = 2

NS = 16
EPT = E // NS
EC = 128
NCHUNK = EPT // EC
ETAIL = EPT - NCHUNK * EC
NSLOT = 2
NGROUP = NCHUNK // NSLOT
RPT = 624
RC = 16

BLK = 1000


def _spmv2_body(g2, src2, dst, zrows, out2,
                src_c, dst_c, rows, src_t, dst_t, rows_t, acc,
                isems, dsems, gsems, csems):
    c = lax.axis_index("c")
    s = lax.axis_index("s")
    row0 = s * RPT
    ebase = c * E + s * EPT
    dbase = s * EPT
    tail = NS * RPT

    def sstart(k, j):
        pltpu.make_async_copy(
            src2.at[pl.ds(ebase + k * EC, EC)], src_c[j], isems[j]).start()

    def swait(j):
        pltpu.make_async_copy(src2.at[pl.ds(0, EC)], src_c[j], isems[j]).wait()

    def dstart(k, j):
        pltpu.make_async_copy(
            dst.at[pl.ds(dbase + k * EC, EC)], dst_c[j], dsems[j]).start()

    def dwait(j):
        pltpu.make_async_copy(dst.at[pl.ds(0, EC)], dst_c[j], dsems[j]).wait()

    for j in range(NSLOT):
        sstart(j, j)

    pltpu.sync_copy(zrows, acc.at[pl.ds(row0, RPT)])

    @pl.when(s == NS - 1)
    def _():
        pltpu.sync_copy(zrows.at[pl.ds(0, N - NS * RPT)],
                        acc.at[pl.ds(tail, N - NS * RPT)])

    plsc.subcore_barrier()

    def gstart(j):
        pltpu.make_async_copy(g2.at[src_c[j]], rows[j], gsems[j]).start()

    def gwait(j):
        pltpu.make_async_copy(g2.at[src_c[j]], rows[j], gsems[j]).wait()

    def cstart(j):
        pltpu.async_copy(rows[j], acc.at[dst_c[j]], csems[j], add=True)

    def cwait(j):
        pltpu.make_async_copy(rows[j], acc.at[dst_c[j]], csems[j]).wait()

    def body(i, carry):
        k0 = NSLOT * i
        dstart(k0, 0)
        swait(0)
        gstart(0)

        @pl.when(i > 0)
        def _():
            cwait(1)

        dstart(k0 + 1, 1)
        swait(1)
        gstart(1)

        gwait(0)

        @pl.when(i < NGROUP - 1)
        def _():
            sstart(k0 + NSLOT, 0)

        dwait(0)
        cstart(0)

        gwait(1)

        @pl.when(i < NGROUP - 1)
        def _():
            sstart(k0 + NSLOT + 1, 1)

        dwait(1)
        cwait(0)
        cstart(1)
        return carry

    lax.fori_loop(0, NGROUP, body, 0)
    cwait(1)
    pltpu.sync_copy(src2.at[pl.ds(ebase + NCHUNK * EC, ETAIL)], src_t)
    pltpu.sync_copy(dst.at[pl.ds(dbase + NCHUNK * EC, ETAIL)], dst_t)
    pltpu.async_copy(g2.at[src_t], rows_t, gsems[0]).wait()
    pltpu.sync_copy(rows_t, acc.at[dst_t], add=True)
    plsc.subcore_barrier()
    ob = c * N + row0
    pltpu.sync_copy(acc.at[pl.ds(row0, RPT)], out2.at[pl.ds(ob, RPT)])

    @pl.when(s == NS - 1)
    def _():
        pltpu.sync_copy(acc.at[pl.ds(tail, N - NS * RPT)],
                        out2.at[pl.ds(c * N + tail, N - NS * RPT)])


_spmv2 = pl.kernel(
    _spmv2_body,
    out_type=jax.ShapeDtypeStruct((2 * N, H), jnp.float32),
    mesh=plsc.VectorSubcoreMesh(core_axis_name="c", subcore_axis_name="s"),
    scratch_types=[
        [pltpu.VMEM((EC,), jnp.int32) for _ in range(NSLOT)],
        [pltpu.VMEM((EC,), jnp.int32) for _ in range(NSLOT)],
        [pltpu.VMEM((EC, H), jnp.float32) for _ in range(NSLOT)],
        pltpu.VMEM((ETAIL,), jnp.int32),
        pltpu.VMEM((ETAIL,), jnp.int32),
        pltpu.VMEM((ETAIL, H), jnp.float32),
        pltpu.VMEM_SHARED((N, H), jnp.float32),
        [pltpu.SemaphoreType.DMA for _ in range(NSLOT)],
        [pltpu.SemaphoreType.DMA for _ in range(NSLOT)],
        [pltpu.SemaphoreType.DMA for _ in range(NSLOT)],
        [pltpu.SemaphoreType.DMA for _ in range(NSLOT)],
    ],
)


def _lift_body(x_ref, w_ref, b_ref, o_ref):
    o_ref[...] = jnp.tanh(
        jnp.dot(x_ref[...], w_ref[...], preferred_element_type=jnp.float32)
        + b_ref[...])


def _readout_body(x_ref, w_ref, b_ref, o_ref):
    o_ref[...] = (
        jnp.dot(x_ref[...], w_ref[...], preferred_element_type=jnp.float32)
        + b_ref[...])


def _stage_a_body(x_ref, y_ref, d_ref, r2_ref, g2_ref):
    x = x_ref[...]
    y = y_ref[...]
    d = d_ref[...]
    rx = x + DT * (x * (1.0 - y))
    ry = y + DT * (y * (x - 1.0))
    r2_ref[0] = rx
    r2_ref[1] = ry
    g2_ref[0] = d * rx
    g2_ref[1] = d * ry


def _stage_b_body(g2_ref, q2_ref, d_ref, w2_ref):
    d = d_ref[...]
    scale = 1.0 / (1.0 + DT)
    dd = d * d
    w2_ref[0] = (g2_ref[0] + DT * dd * q2_ref[0]) * scale
    w2_ref[1] = (g2_ref[1] + DT * dd * q2_ref[1]) * scale


def _stage_c_body(tau_ref, x_ref, y_ref, r2_ref, q2_ref, d_ref,
                  xo_ref, yo_ref):
    d = d_ref[...]
    tau = tau_ref[0]
    scale = 1.0 / (1.0 + DT)
    zx = (r2_ref[0] + DT * d * q2_ref[0]) * scale
    zy = (r2_ref[1] + DT * d * q2_ref[1]) * scale
    xo_ref[...] = (1.0 - tau) * x_ref[...] + tau * zx
    yo_ref[...] = (1.0 - tau) * y_ref[...] + tau * zy


def _stage_ca_body(tau_ref, x_ref, y_ref, r2_ref, q2_ref, d_ref,
                   xo_ref, yo_ref, r2o_ref, g2o_ref):
    d = d_ref[...]
    tau = tau_ref[0]
    scale = 1.0 / (1.0 + DT)
    zx = (r2_ref[0] + DT * d * q2_ref[0]) * scale
    zy = (r2_ref[1] + DT * d * q2_ref[1]) * scale
    x = (1.0 - tau) * x_ref[...] + tau * zx
    y = (1.0 - tau) * y_ref[...] + tau * zy
    xo_ref[...] = x
    yo_ref[...] = y
    rx = x + DT * (x * (1.0 - y))
    ry = y + DT * (y * (x - 1.0))
    r2o_ref[0] = rx
    r2o_ref[1] = ry
    g2o_ref[0] = d * rx
    g2o_ref[1] = d * ry


_GRID = N // BLK
_row_spec = pl.BlockSpec((BLK, H), lambda i: (i, 0))
_pair_spec = pl.BlockSpec((2, BLK, H), lambda i: (0, i, 0))
_full_spec = pl.BlockSpec((H, H), lambda i: (0, 0))
_brow_spec = pl.BlockSpec((1, H), lambda i: (0, 0))

_lift = pl.pallas_call(
    _lift_body,
    grid=(_GRID,),
    in_specs=[_row_spec, _full_spec, _brow_spec],
    out_specs=_row_spec,
    out_shape=jax.ShapeDtypeStruct((N, H), jnp.float32),
)

_readout = pl.pallas_call(
    _readout_body,
    grid=(_GRID,),
    in_specs=[_row_spec, _full_spec, _brow_spec],
    out_specs=_row_spec,
    out_shape=jax.ShapeDtypeStruct((N, H), jnp.float32),
)

_stage_a = pl.pallas_call(
    _stage_a_body,
    grid=(_GRID,),
    in_specs=[_row_spec, _row_spec, _row_spec],
    out_specs=[_pair_spec, _pair_spec],
    out_shape=[jax.ShapeDtypeStruct((2, N, H), jnp.float32),
               jax.ShapeDtypeStruct((2, N, H), jnp.float32)],
)

_stage_b = pl.pallas_call(
    _stage_b_body,
    grid=(_GRID,),
    in_specs=[_pair_spec, _pair_spec, _row_spec],
    out_specs=_pair_spec,
    out_shape=jax.ShapeDtypeStruct((2, N, H), jnp.float32),
)

_stage_c = pl.pallas_call(
    _stage_c_body,
    grid=(_GRID,),
    in_specs=[pl.BlockSpec(memory_space=pltpu.SMEM),
              _row_spec, _row_spec, _pair_spec, _pair_spec, _row_spec],
    out_specs=[_row_spec, _row_spec],
    out_shape=[jax.ShapeDtypeStruct((N, H), jnp.float32),
               jax.ShapeDtypeStruct((N, H), jnp.float32)],
)

_stage_ca = pl.pallas_call(
    _stage_ca_body,
    grid=(_GRID,),
    in_specs=[pl.BlockSpec(memory_space=pltpu.SMEM),
              _row_spec, _row_spec, _pair_spec, _pair_spec, _row_spec],
    out_specs=[_row_spec, _row_spec, _pair_spec, _pair_spec],
    out_shape=[jax.ShapeDtypeStruct((N, H), jnp.float32),
               jax.ShapeDtypeStruct((N, H), jnp.float32),
               jax.ShapeDtypeStruct((2, N, H), jnp.float32),
               jax.ShapeDtypeStruct((2, N, H), jnp.float32)],
)


def kernel(x, edge_index, lift_x_w, lift_x_b, taus, logit_scale,
           readout_w, readout_b):
    src = edge_index[0]
    dst = edge_index[1]
    src2 = jnp.concatenate([src, src + N])


    ones2 = jnp.ones((2 * N, H), dtype=jnp.float32)
    zrows = jnp.zeros((RPT, H), dtype=jnp.float32)
    deg = _spmv2(ones2, src2, dst, zrows)[:N, :1]
    dinv = 1.0 / jnp.sqrt(jnp.maximum(deg, 1.0))
    dinvb = jnp.broadcast_to(dinv, (N, H))

    X = _lift(x, lift_x_w, lift_x_b.reshape(1, H))
    Y = jnp.ones_like(X)

    sig_taus = jax.nn.sigmoid(taus)

    r2, g2 = _stage_a(X, Y, dinvb)
    for li in range(LAYERS):
        q2 = _spmv2(g2.reshape(2 * N, H), src2, dst, zrows)
        w2 = _stage_b(g2, q2.reshape(2, N, H), dinvb)
        q2b = _spmv2(w2.reshape(2 * N, H), src2, dst, zrows)
        if li < LAYERS - 1:
            X, Y, r2, g2 = _stage_ca(sig_taus[li].reshape(1), X, Y, r2,
                                     q2b.reshape(2, N, H), dinvb)
        else:
            X, Y = _stage_c(sig_taus[li].reshape(1), X, Y, r2,
                            q2b.reshape(2, N, H), dinvb)

    wp = jnp.zeros((H, H), jnp.float32).at[:, :CLS].set(
        logit_scale * readout_w)
    bp = jnp.zeros((1, H), jnp.float32).at[0, :CLS].set(readout_b)
    out = _readout(X, wp, bp)
    return out[:, :CLS]


# speedup vs baseline: 1.7039x; 1.0043x over previous
import jax
import jax.numpy as jnp
from jax import lax
from jax.experimental import pallas as pl
from jax.experimental.pallas import tpu as pltpu
from jax.experimental.pallas import tpu_sc as plsc

N = 10000
E = 320000
H = 128
CLS = 40
LAYERS = 15
DT = 0.1

NS = 16
EPT = E // NS
EC = 128
NCHUNK = EPT // EC
ETAIL = EPT - NCHUNK * EC
NSLOT = 2
NGROUP = NCHUNK // NSLOT
RPT = 624

BLK = 1000


def _spmv2_body(g2, src2, dst, zrows, out2,
                src_c, dst_c, rows, src_t, dst_t, rows_t, acc,
                isems, dsems, gsems, csems):
    c = lax.axis_index("c")
    s = lax.axis_index("s")
    row0 = s * RPT
    ebase = c * E + s * EPT
    dbase = s * EPT
    tail = NS * RPT

    def sstart(k, j):
        pltpu.make_async_copy(
            src2.at[pl.ds(ebase + k * EC, EC)], src_c[j], isems[j]).start()

    def swait(j):
        pltpu.make_async_copy(src2.at[pl.ds(0, EC)], src_c[j], isems[j]).wait()

    def dstart(k, j):
        pltpu.make_async_copy(
            dst.at[pl.ds(dbase + k * EC, EC)], dst_c[j], dsems[j]).start()

    def dwait(j):
        pltpu.make_async_copy(dst.at[pl.ds(0, EC)], dst_c[j], dsems[j]).wait()

    for j in range(NSLOT):
        sstart(j, j)

    pltpu.sync_copy(zrows, acc.at[pl.ds(row0, RPT)])

    @pl.when(s == NS - 1)
    def _():
        pltpu.sync_copy(zrows.at[pl.ds(0, N - NS * RPT)],
                        acc.at[pl.ds(tail, N - NS * RPT)])

    plsc.subcore_barrier()

    def gstart(j):
        pltpu.make_async_copy(g2.at[src_c[j]], rows[j], gsems[j]).start()

    def gwait(j):
        pltpu.make_async_copy(g2.at[src_c[j]], rows[j], gsems[j]).wait()

    def cstart(j):
        pltpu.async_copy(rows[j], acc.at[dst_c[j]], csems[j], add=True)

    def cwait(j):
        pltpu.make_async_copy(rows[j], acc.at[dst_c[j]], csems[j]).wait()

    def body(i, carry):
        k0 = NSLOT * i
        dstart(k0, 0)
        swait(0)
        gstart(0)

        @pl.when(i > 0)
        def _():
            cwait(1)

        dstart(k0 + 1, 1)
        swait(1)
        gstart(1)

        gwait(0)

        @pl.when(i < NGROUP - 1)
        def _():
            sstart(k0 + NSLOT, 0)

        dwait(0)
        cstart(0)

        gwait(1)

        @pl.when(i < NGROUP - 1)
        def _():
            sstart(k0 + NSLOT + 1, 1)

        dwait(1)
        cwait(0)
        cstart(1)
        return carry

    lax.fori_loop(0, NGROUP, body, 0)
    cwait(1)
    pltpu.sync_copy(src2.at[pl.ds(ebase + NCHUNK * EC, ETAIL)], src_t)
    pltpu.sync_copy(dst.at[pl.ds(dbase + NCHUNK * EC, ETAIL)], dst_t)
    pltpu.async_copy(g2.at[src_t], rows_t, gsems[0]).wait()
    pltpu.sync_copy(rows_t, acc.at[dst_t], add=True)
    plsc.subcore_barrier()
    ob = c * N + row0
    pltpu.sync_copy(acc.at[pl.ds(row0, RPT)], out2.at[pl.ds(ob, RPT)])

    @pl.when(s == NS - 1)
    def _():
        pltpu.sync_copy(acc.at[pl.ds(tail, N - NS * RPT)],
                        out2.at[pl.ds(c * N + tail, N - NS * RPT)])


_spmv2 = pl.kernel(
    _spmv2_body,
    out_type=jax.ShapeDtypeStruct((2 * N, H), jnp.float32),
    mesh=plsc.VectorSubcoreMesh(core_axis_name="c", subcore_axis_name="s",
                                num_cores=2, num_subcores=NS),
    scratch_types=[
        [pltpu.VMEM((EC,), jnp.int32) for _ in range(NSLOT)],
        [pltpu.VMEM((EC,), jnp.int32) for _ in range(NSLOT)],
        [pltpu.VMEM((EC, H), jnp.float32) for _ in range(NSLOT)],
        pltpu.VMEM((ETAIL,), jnp.int32),
        pltpu.VMEM((ETAIL,), jnp.int32),
        pltpu.VMEM((ETAIL, H), jnp.float32),
        pltpu.VMEM_SHARED((N, H), jnp.float32),
        [pltpu.SemaphoreType.DMA for _ in range(NSLOT)],
        [pltpu.SemaphoreType.DMA for _ in range(NSLOT)],
        [pltpu.SemaphoreType.DMA for _ in range(NSLOT)],
        [pltpu.SemaphoreType.DMA for _ in range(NSLOT)],
    ],
)


def _lift_body(x_ref, w_ref, b_ref, o_ref):
    o_ref[...] = jnp.tanh(
        jnp.dot(x_ref[...], w_ref[...], preferred_element_type=jnp.float32)
        + b_ref[...])


def _readout_body(x_ref, w_ref, b_ref, o_ref):
    o_ref[...] = (
        jnp.dot(x_ref[...], w_ref[...], preferred_element_type=jnp.float32)
        + b_ref[...])


def _stage_a_body(x_ref, y_ref, d_ref, r2_ref, g2_ref):
    x = x_ref[...]
    y = y_ref[...]
    d = d_ref[...]
    rx = x + DT * (x * (1.0 - y))
    ry = y + DT * (y * (x - 1.0))
    r2_ref[0] = rx
    r2_ref[1] = ry
    g2_ref[0] = d * rx
    g2_ref[1] = d * ry


def _stage_b_body(g2_ref, q2_ref, d_ref, w2_ref):
    d = d_ref[...]
    scale = 1.0 / (1.0 + DT)
    dd = d * d
    w2_ref[0] = (g2_ref[0] + DT * dd * q2_ref[0]) * scale
    w2_ref[1] = (g2_ref[1] + DT * dd * q2_ref[1]) * scale


def _stage_c_body(tau_ref, x_ref, y_ref, r2_ref, q2_ref, d_ref,
                  xo_ref, yo_ref):
    d = d_ref[...]
    tau = tau_ref[0]
    scale = 1.0 / (1.0 + DT)
    zx = (r2_ref[0] + DT * d * q2_ref[0]) * scale
    zy = (r2_ref[1] + DT * d * q2_ref[1]) * scale
    xo_ref[...] = (1.0 - tau) * x_ref[...] + tau * zx
    yo_ref[...] = (1.0 - tau) * y_ref[...] + tau * zy


def _stage_ca_body(tau_ref, x_ref, y_ref, r2_ref, q2_ref, d_ref,
                   xo_ref, yo_ref, r2o_ref, g2o_ref):
    d = d_ref[...]
    tau = tau_ref[0]
    scale = 1.0 / (1.0 + DT)
    zx = (r2_ref[0] + DT * d * q2_ref[0]) * scale
    zy = (r2_ref[1] + DT * d * q2_ref[1]) * scale
    x = (1.0 - tau) * x_ref[...] + tau * zx
    y = (1.0 - tau) * y_ref[...] + tau * zy
    xo_ref[...] = x
    yo_ref[...] = y
    rx = x + DT * (x * (1.0 - y))
    ry = y + DT * (y * (x - 1.0))
    r2o_ref[0] = rx
    r2o_ref[1] = ry
    g2o_ref[0] = d * rx
    g2o_ref[1] = d * ry


_GRID = N // BLK
_row_spec = pl.BlockSpec((BLK, H), lambda i: (i, 0))
_pair_spec = pl.BlockSpec((2, BLK, H), lambda i: (0, i, 0))
_full_spec = pl.BlockSpec((H, H), lambda i: (0, 0))
_brow_spec = pl.BlockSpec((1, H), lambda i: (0, 0))

_lift = pl.pallas_call(
    _lift_body,
    grid=(_GRID,),
    in_specs=[_row_spec, _full_spec, _brow_spec],
    out_specs=_row_spec,
    out_shape=jax.ShapeDtypeStruct((N, H), jnp.float32),
)

_readout = pl.pallas_call(
    _readout_body,
    grid=(_GRID,),
    in_specs=[_row_spec, _full_spec, _brow_spec],
    out_specs=_row_spec,
    out_shape=jax.ShapeDtypeStruct((N, H), jnp.float32),
)

_stage_a = pl.pallas_call(
    _stage_a_body,
    grid=(_GRID,),
    in_specs=[_row_spec, _row_spec, _row_spec],
    out_specs=[_pair_spec, _pair_spec],
    out_shape=[jax.ShapeDtypeStruct((2, N, H), jnp.float32),
               jax.ShapeDtypeStruct((2, N, H), jnp.float32)],
)

_stage_b = pl.pallas_call(
    _stage_b_body,
    grid=(_GRID,),
    in_specs=[_pair_spec, _pair_spec, _row_spec],
    out_specs=_pair_spec,
    out_shape=jax.ShapeDtypeStruct((2, N, H), jnp.float32),
)

_stage_c = pl.pallas_call(
    _stage_c_body,
    grid=(_GRID,),
    in_specs=[pl.BlockSpec(memory_space=pltpu.SMEM),
              _row_spec, _row_spec, _pair_spec, _pair_spec, _row_spec],
    out_specs=[_row_spec, _row_spec],
    out_shape=[jax.ShapeDtypeStruct((N, H), jnp.float32),
               jax.ShapeDtypeStruct((N, H), jnp.float32)],
)

_stage_ca = pl.pallas_call(
    _stage_ca_body,
    grid=(_GRID,),
    in_specs=[pl.BlockSpec(memory_space=pltpu.SMEM),
              _row_spec, _row_spec, _pair_spec, _pair_spec, _row_spec],
    out_specs=[_row_spec, _row_spec, _pair_spec, _pair_spec],
    out_shape=[jax.ShapeDtypeStruct((N, H), jnp.float32),
               jax.ShapeDtypeStruct((N, H), jnp.float32),
               jax.ShapeDtypeStruct((2, N, H), jnp.float32),
               jax.ShapeDtypeStruct((2, N, H), jnp.float32)],
)


def kernel(x, edge_index, lift_x_w, lift_x_b, taus, logit_scale,
           readout_w, readout_b):
    src = edge_index[0]
    dst = edge_index[1]
    src2 = jnp.concatenate([src, src + N])


    ones2 = jnp.ones((2 * N, H), dtype=jnp.float32)
    zrows = jnp.zeros((RPT, H), dtype=jnp.float32)
    deg = _spmv2(ones2, src2, dst, zrows)[:N, :1]
    dinv = 1.0 / jnp.sqrt(jnp.maximum(deg, 1.0))
    dinvb = jnp.broadcast_to(dinv, (N, H))

    X = _lift(x, lift_x_w, lift_x_b.reshape(1, H))
    Y = jnp.ones_like(X)

    sig_taus = jax.nn.sigmoid(taus)

    r2, g2 = _stage_a(X, Y, dinvb)
    for li in range(LAYERS):
        q2 = _spmv2(g2.reshape(2 * N, H), src2, dst, zrows)
        w2 = _stage_b(g2, q2.reshape(2, N, H), dinvb)
        q2b = _spmv2(w2.reshape(2 * N, H), src2, dst, zrows)
        if li < LAYERS - 1:
            X, Y, r2, g2 = _stage_ca(sig_taus[li].reshape(1), X, Y, r2,
                                     q2b.reshape(2, N, H), dinvb)
        else:
            X, Y = _stage_c(sig_taus[li].reshape(1), X, Y, r2,
                            q2b.reshape(2, N, H), dinvb)

    wp = jnp.zeros((H, H), jnp.float32).at[:, :CLS].set(
        logit_scale * readout_w)
    bp = jnp.zeros((1, H), jnp.float32).at[0, :CLS].set(readout_b)
    out = _readout(X, wp, bp)
    return out[:, :CLS]
